# baseline (speedup 1.0000x reference)
import functools

import jax
import jax.numpy as jnp
from jax import lax
from jax.experimental import pallas as pl
from jax.experimental.pallas import tpu as pltpu
from jax.experimental.pallas import tpu_sc as plsc

N = 10000
NP = 10240
DUMP = 10000
HID = 128
HEADS = 8
OUT = 16
ROWW = 144
NGROUPS = 64
RB = 1024
K = 56
EDGE_ALIGN = 64 * K


def _proj_body(h_blk, Ws_ref, As_ref, Ad_ref, xl_ref, ts_ref, td_ref, g_ref):
    i = pl.program_id(0)
    xl = jnp.dot(h_blk, Ws_ref[...], preferred_element_type=jnp.float32)
    xl_ref[...] = xl
    ts = jnp.dot(xl, As_ref[...], preferred_element_type=jnp.float32)
    td = jnp.dot(xl, Ad_ref[...], preferred_element_type=jnp.float32)
    ts_ref[...] = ts
    td_ref[...] = td
    bm = jnp.max(ts, axis=0, keepdims=True)
    bm = jnp.concatenate([bm, jnp.full((1, 112), -1e30, jnp.float32)], axis=1)

    @pl.when(i == 0)
    def _():
        g_ref[...] = jnp.full((1, 128), -1e30, jnp.float32)

    g_ref[...] = jnp.maximum(g_ref[...], bm)


def _tc_first(x_ref, W0_ref, b0_ref, Ws_ref, As_ref, Ad_ref,
              xl_ref, ts_ref, td_ref, g_ref):
    h = jnp.dot(x_ref[...], W0_ref[...], preferred_element_type=jnp.float32)
    h = h + b0_ref[...]
    _proj_body(h, Ws_ref, As_ref, Ad_ref, xl_ref, ts_ref, td_ref, g_ref)


def _h_from_partials(p0, p1, bias):
    unn = p0[:, :128] + p1[:, :128]
    den = p0[:, 128:136] + p1[:, 128:136]
    den = jnp.broadcast_to(den[:, :, None], (den.shape[0], 8, 16))
    den = den.reshape(den.shape[0], 128)
    h = unn / (den + 1e-16) + bias
    return jnp.maximum(h, 0.0)


def _tc_mid(p0_ref, p1_ref, b_ref, Ws_ref, As_ref, Ad_ref,
            xl_ref, ts_ref, td_ref, g_ref):
    h = _h_from_partials(p0_ref[...], p1_ref[...], b_ref[...])
    _proj_body(h, Ws_ref, As_ref, Ad_ref, xl_ref, ts_ref, td_ref, g_ref)


def _tc_final(p0_ref, p1_ref, b_ref, batch_ref, Wfc_ref, bfc_ref,
              pooled_ref, cnt_ref, res_ref):
    i = pl.program_id(0)
    h = _h_from_partials(p0_ref[...], p1_ref[...], b_ref[...])
    ids = batch_ref[0]
    gids = lax.broadcasted_iota(jnp.int32, (NGROUPS, RB), 0)
    onehot = (ids == gids).astype(jnp.float32)

    @pl.when(i == 0)
    def _():
        pooled_ref[...] = jnp.zeros((NGROUPS, 128), jnp.float32)
        cnt_ref[...] = jnp.zeros((NGROUPS, 128), jnp.float32)

    pooled_ref[...] += jnp.dot(onehot, h, preferred_element_type=jnp.float32)
    cnt_ref[...] += jnp.dot(onehot, jnp.ones((RB, 128), jnp.float32),
                            preferred_element_type=jnp.float32)

    @pl.when(i == pl.num_programs(0) - 1)
    def _():
        pooled = pooled_ref[...] / jnp.maximum(cnt_ref[...], 1.0)
        res_ref[...] = jnp.dot(pooled, Wfc_ref[...],
                               preferred_element_type=jnp.float32) + bfc_ref[...]


def _row_spec(w):
    return pl.BlockSpec((RB, w), lambda i: (i, 0))


def _full_spec(shape):
    return pl.BlockSpec(shape, lambda i: tuple(0 for _ in shape))


def _tc_first_call(xp, W0, b0, Ws0, As0, Ad0):
    grid = NP // RB
    return pl.pallas_call(
        _tc_first,
        grid=(grid,),
        in_specs=[_row_spec(128), _full_spec((128, 128)), _full_spec((1, 128)),
                  _full_spec((128, 128)), _full_spec((128, 16)),
                  _full_spec((128, 16))],
        out_specs=[_row_spec(128), _row_spec(16), _row_spec(16),
                   _full_spec((1, 128))],
        out_shape=[jax.ShapeDtypeStruct((NP, 128), jnp.float32),
                   jax.ShapeDtypeStruct((NP, 16), jnp.float32),
                   jax.ShapeDtypeStruct((NP, 16), jnp.float32),
                   jax.ShapeDtypeStruct((1, 128), jnp.float32)],
    )(xp, W0, b0, Ws0, As0, Ad0)


def _tc_mid_call(p0, p1, bias, Wsi, Asi, Adi):
    grid = NP // RB
    return pl.pallas_call(
        _tc_mid,
        grid=(grid,),
        in_specs=[_row_spec(ROWW), _row_spec(ROWW), _full_spec((1, 128)),
                  _full_spec((128, 128)), _full_spec((128, 16)),
                  _full_spec((128, 16))],
        out_specs=[_row_spec(128), _row_spec(16), _row_spec(16),
                   _full_spec((1, 128))],
        out_shape=[jax.ShapeDtypeStruct((NP, 128), jnp.float32),
                   jax.ShapeDtypeStruct((NP, 16), jnp.float32),
                   jax.ShapeDtypeStruct((NP, 16), jnp.float32),
                   jax.ShapeDtypeStruct((1, 128), jnp.float32)],
    )(p0, p1, bias, Wsi, Asi, Adi)


def _tc_final_call(p0, p1, bias, batch3, Wfc, bfc):
    grid = NP // RB
    res = pl.pallas_call(
        _tc_final,
        grid=(grid,),
        in_specs=[_row_spec(ROWW), _row_spec(ROWW), _full_spec((1, 128)),
                  pl.BlockSpec((1, 1, RB), lambda i: (i, 0, 0)),
                  _full_spec((128, 128)), _full_spec((1, 128))],
        out_specs=[_full_spec((NGROUPS, 128)), _full_spec((NGROUPS, 128)),
                   _full_spec((NGROUPS, 128))],
        out_shape=[jax.ShapeDtypeStruct((NGROUPS, 128), jnp.float32),
                   jax.ShapeDtypeStruct((NGROUPS, 128), jnp.float32),
                   jax.ShapeDtypeStruct((NGROUPS, 128), jnp.float32)],
    )(p0, p1, bias, batch3, Wfc, bfc)
    return res[2]


def _leaky(v):
    return jnp.where(v >= 0.0, v, 0.2 * v)


def _make_sc_edge(etot_pad):
    chunks_per_worker = etot_pad // (32 * K)
    edges_per_worker = chunks_per_worker * K
    rows_per_tile = NP // 16
    n_zero_full = rows_per_tile // K
    zero_rem = rows_per_tile - n_zero_full * K
    if chunks_per_worker % 18 == 0:
        group = 18
    elif chunks_per_worker % 6 == 0:
        group = 6
    else:
        group = 2
    n_groups = chunks_per_worker // group

    mesh = plsc.VectorSubcoreMesh(core_axis_name="c", subcore_axis_name="s")

    @functools.partial(
        pl.kernel,
        out_type=jax.ShapeDtypeStruct((2, NP, ROWW), jnp.float32),
        mesh=mesh,
        compiler_params=pltpu.CompilerParams(use_tc_tiling_on_sc=False),
        scratch_types=[
            pltpu.VMEM((group, K), jnp.int32),
            pltpu.VMEM((group, K), jnp.int32),
            pltpu.VMEM((K, 16), jnp.float32), pltpu.VMEM((K, 16), jnp.float32),
            pltpu.VMEM((K, 16), jnp.float32), pltpu.VMEM((K, 16), jnp.float32),
            pltpu.VMEM((K, 128), jnp.float32), pltpu.VMEM((K, 128), jnp.float32),
            pltpu.VMEM((K, ROWW), jnp.float32),
            pltpu.VMEM((K, ROWW), jnp.float32),
            pltpu.VMEM((16,), jnp.float32),
            pltpu.VMEM_SHARED((NP, ROWW), jnp.float32),
            pltpu.SemaphoreType.DMA, pltpu.SemaphoreType.DMA,
            pltpu.SemaphoreType.DMA, pltpu.SemaphoreType.DMA,
        ],
    )
    def sc_edge(src_hbm, dst_hbm, ts_hbm, td_hbm, xl_hbm, g_hbm, out_hbm,
                sbuf, dbuf, srows0, srows1, drows0, drows1,
                xrows0, xrows1, stag0, stag1, gv, acc,
                sem0, sem1, scsem0, scsem1):
        cid = lax.axis_index("c")
        sid = lax.axis_index("s")
        wid = sid * 2 + cid
        base_row = sid * rows_per_tile
        base_chunk = wid * chunks_per_worker
        srows = (srows0, srows1)
        drows = (drows0, drows1)
        xrows = (xrows0, xrows1)
        stag = (stag0, stag1)
        sem = (sem0, sem1)
        scsem = (scsem0, scsem1)

        @plsc.parallel_loop(0, K, unroll=4)
        def _(r):
            for cc in range(ROWW // 16):
                stag0[r, cc * 16:(cc + 1) * 16] = jnp.zeros((16,), jnp.float32)

        for j in range(n_zero_full):
            pltpu.sync_copy(stag0, acc.at[pl.ds(base_row + j * K, K)])
        if zero_rem:
            pltpu.sync_copy(stag0.at[pl.ds(0, zero_rem)],
                            acc.at[pl.ds(base_row + n_zero_full * K, zero_rem)])
        pltpu.sync_copy(g_hbm, gv)
        plsc.subcore_barrier()

        def launch(j, b):
            pltpu.async_copy(ts_hbm.at[sbuf.at[j]], srows[b], sem[b])
            pltpu.async_copy(td_hbm.at[dbuf.at[j]], drows[b], sem[b])
            pltpu.async_copy(xl_hbm.at[sbuf.at[j]], xrows[b], sem[b])

        def drain(j, b):
            pltpu.make_async_copy(ts_hbm.at[sbuf.at[j]], srows[b], sem[b]).wait()
            pltpu.make_async_copy(td_hbm.at[dbuf.at[j]], drows[b], sem[b]).wait()
            pltpu.make_async_copy(xl_hbm.at[sbuf.at[j]], xrows[b], sem[b]).wait()

        def compute(j, b):
            gvec = gv[...]

            @pl.when(j >= 2)
            def _():
                pltpu.make_async_copy(stag[b], acc.at[dbuf.at[j]],
                                      scsem[b]).wait()

            @plsc.parallel_loop(0, K, unroll=4)
            def _(e):
                sv = srows[b][e, 0:16]
                dv = drows[b][e, 0:16]
                w = jnp.exp(_leaky(sv + dv) - _leaky(gvec + dv))
                stag[b][e, 128:144] = w
                for h in range(HEADS):
                    stag[b][e, h * 16:(h + 1) * 16] = (
                        xrows[b][e, h * 16:(h + 1) * 16] * w[h])

            pltpu.async_copy(stag[b], acc.at[dbuf.at[j]], scsem[b], add=True)

        def group_body(gr, _):
            @pl.when(gr >= 1)
            def _():
                pltpu.make_async_copy(stag0, acc.at[dbuf.at[0]], scsem0).wait()
                pltpu.make_async_copy(stag1, acc.at[dbuf.at[0]], scsem1).wait()

            gbase = base_chunk + gr * group
            pltpu.sync_copy(src_hbm.at[pl.ds(gbase, group)], sbuf)
            pltpu.sync_copy(dst_hbm.at[pl.ds(gbase, group)], dbuf)
            launch(0, 0)

            def pair(jp, _):
                j0 = jp * 2
                launch(j0 + 1, 1)
                drain(j0, 0)
                compute(j0, 0)

                @pl.when(j0 + 2 < group)
                def _():
                    launch(j0 + 2, 0)

                drain(j0 + 1, 1)
                compute(j0 + 1, 1)
                return 0

            lax.fori_loop(0, group // 2, pair, 0)
            return 0

        lax.fori_loop(0, n_groups, group_body, 0)

        pltpu.make_async_copy(stag0, acc.at[dbuf.at[group - 2]], scsem0).wait()
        pltpu.make_async_copy(stag1, acc.at[dbuf.at[group - 1]], scsem1).wait()

        plsc.subcore_barrier()
        pltpu.sync_copy(acc.at[pl.ds(base_row, rows_per_tile)],
                        out_hbm.at[cid].at[pl.ds(base_row, rows_per_tile)])

    return sc_edge


def kernel(x, edge_index, batch, W0, b0, Ws, att_src, att_dst, conv_bias,
           Wfc, bfc):
    L = Ws.shape[0]
    n = x.shape[0]

    xp = jnp.pad(x, ((0, NP - n), (0, 0)))
    loop = jnp.arange(n, dtype=edge_index.dtype)
    src = jnp.concatenate([edge_index[0], loop])
    dst = jnp.concatenate([edge_index[1], loop])
    etot = src.shape[0]
    etot_pad = ((etot + EDGE_ALIGN - 1) // EDGE_ALIGN) * EDGE_ALIGN
    src = jnp.pad(src, (0, etot_pad - etot)).reshape(-1, K)
    dst = jnp.pad(dst, (0, etot_pad - etot),
                  constant_values=DUMP).reshape(-1, K)
    batch3 = jnp.pad(batch, (0, NP - n), constant_values=-1).reshape(
        NP // RB, 1, RB)

    eye = jnp.eye(HEADS, dtype=jnp.float32)
    As = (att_src[:, :, :, None] * eye[:, None, :]).reshape(L, HID, HEADS)
    Ad = (att_dst[:, :, :, None] * eye[:, None, :]).reshape(L, HID, HEADS)
    As = jnp.pad(As, ((0, 0), (0, 0), (0, 8)))
    Ad = jnp.pad(Ad, ((0, 0), (0, 0), (0, 8)))
    b0r = b0.reshape(1, HID)
    biasr = conv_bias.reshape(L, 1, HID)
    bfcr = bfc.reshape(1, -1)

    sc_edge = _make_sc_edge(etot_pad)

    def g_fix(g_out):
        g16 = g_out[0, :16]
        return jnp.where(jnp.arange(16) < 8, g16, 1e9).astype(jnp.float32)

    xl, ts, td, g = _tc_first_call(xp, W0, b0r, Ws[0], As[0], Ad[0])
    parts = sc_edge(src, dst, ts, td, xl, g_fix(g))
    for i in range(1, L):
        xl, ts, td, g = _tc_mid_call(parts[0], parts[1], biasr[i - 1],
                                     Ws[i], As[i], Ad[i])
        parts = sc_edge(src, dst, ts, td, xl, g_fix(g))
    return _tc_final_call(parts[0], parts[1], biasr[L - 1], batch3, Wfc, bfcr)


# speedup vs baseline: 1.3815x; 1.3815x over previous
import functools

import jax
import jax.numpy as jnp
from jax import lax
from jax.experimental import pallas as pl
from jax.experimental.pallas import tpu as pltpu
from jax.experimental.pallas import tpu_sc as plsc

N = 10000
NP = 10240
DUMP = 10000
HID = 128
HEADS = 8
OUT = 16
ROWW = 144
NGROUPS = 64
RB = 1024
K = 64
EDGE_ALIGN = 64 * K


def _proj_body(h_blk, Ws_ref, As_ref, Ad_ref, xl_ref, ts_ref, td_ref, g_ref):
    i = pl.program_id(0)
    xl = jnp.dot(h_blk, Ws_ref[...], preferred_element_type=jnp.float32)
    xl_ref[...] = xl
    ts = jnp.dot(xl, As_ref[...], preferred_element_type=jnp.float32)
    td = jnp.dot(xl, Ad_ref[...], preferred_element_type=jnp.float32)
    ts_ref[...] = ts
    td_ref[...] = td
    bm = jnp.max(ts, axis=0, keepdims=True)
    bm = jnp.concatenate([bm, jnp.full((1, 112), -1e30, jnp.float32)], axis=1)

    @pl.when(i == 0)
    def _():
        g_ref[...] = jnp.full((1, 128), -1e30, jnp.float32)

    g_ref[...] = jnp.maximum(g_ref[...], bm)


def _tc_first(x_ref, W0_ref, b0_ref, Ws_ref, As_ref, Ad_ref,
              xl_ref, ts_ref, td_ref, g_ref):
    h = jnp.dot(x_ref[...], W0_ref[...], preferred_element_type=jnp.float32)
    h = h + b0_ref[...]
    _proj_body(h, Ws_ref, As_ref, Ad_ref, xl_ref, ts_ref, td_ref, g_ref)


def _h_from_partials(p0, p1, bias):
    unn = p0[:, :128] + p1[:, :128]
    den = p0[:, 128:136] + p1[:, 128:136]
    den = jnp.broadcast_to(den[:, :, None], (den.shape[0], 8, 16))
    den = den.reshape(den.shape[0], 128)
    h = unn / (den + 1e-16) + bias
    return jnp.maximum(h, 0.0)


def _tc_mid(p0_ref, p1_ref, b_ref, Ws_ref, As_ref, Ad_ref,
            xl_ref, ts_ref, td_ref, g_ref):
    h = _h_from_partials(p0_ref[...], p1_ref[...], b_ref[...])
    _proj_body(h, Ws_ref, As_ref, Ad_ref, xl_ref, ts_ref, td_ref, g_ref)


def _tc_final(p0_ref, p1_ref, b_ref, batch_ref, Wfc_ref, bfc_ref,
              pooled_ref, cnt_ref, res_ref):
    i = pl.program_id(0)
    h = _h_from_partials(p0_ref[...], p1_ref[...], b_ref[...])
    ids = batch_ref[0]
    gids = lax.broadcasted_iota(jnp.int32, (NGROUPS, RB), 0)
    onehot = (ids == gids).astype(jnp.float32)

    @pl.when(i == 0)
    def _():
        pooled_ref[...] = jnp.zeros((NGROUPS, 128), jnp.float32)
        cnt_ref[...] = jnp.zeros((NGROUPS, 128), jnp.float32)

    pooled_ref[...] += jnp.dot(onehot, h, preferred_element_type=jnp.float32)
    cnt_ref[...] += jnp.dot(onehot, jnp.ones((RB, 128), jnp.float32),
                            preferred_element_type=jnp.float32)

    @pl.when(i == pl.num_programs(0) - 1)
    def _():
        pooled = pooled_ref[...] / jnp.maximum(cnt_ref[...], 1.0)
        res_ref[...] = jnp.dot(pooled, Wfc_ref[...],
                               preferred_element_type=jnp.float32) + bfc_ref[...]


def _row_spec(w):
    return pl.BlockSpec((RB, w), lambda i: (i, 0))


def _full_spec(shape):
    return pl.BlockSpec(shape, lambda i: tuple(0 for _ in shape))


def _tc_first_call(xp, W0, b0, Ws0, As0, Ad0):
    grid = NP // RB
    return pl.pallas_call(
        _tc_first,
        grid=(grid,),
        in_specs=[_row_spec(128), _full_spec((128, 128)), _full_spec((1, 128)),
                  _full_spec((128, 128)), _full_spec((128, 16)),
                  _full_spec((128, 16))],
        out_specs=[_row_spec(128), _row_spec(16), _row_spec(16),
                   _full_spec((1, 128))],
        out_shape=[jax.ShapeDtypeStruct((NP, 128), jnp.float32),
                   jax.ShapeDtypeStruct((NP, 16), jnp.float32),
                   jax.ShapeDtypeStruct((NP, 16), jnp.float32),
                   jax.ShapeDtypeStruct((1, 128), jnp.float32)],
    )(xp, W0, b0, Ws0, As0, Ad0)


def _tc_mid_call(p0, p1, bias, Wsi, Asi, Adi):
    grid = NP // RB
    return pl.pallas_call(
        _tc_mid,
        grid=(grid,),
        in_specs=[_row_spec(ROWW), _row_spec(ROWW), _full_spec((1, 128)),
                  _full_spec((128, 128)), _full_spec((128, 16)),
                  _full_spec((128, 16))],
        out_specs=[_row_spec(128), _row_spec(16), _row_spec(16),
                   _full_spec((1, 128))],
        out_shape=[jax.ShapeDtypeStruct((NP, 128), jnp.float32),
                   jax.ShapeDtypeStruct((NP, 16), jnp.float32),
                   jax.ShapeDtypeStruct((NP, 16), jnp.float32),
                   jax.ShapeDtypeStruct((1, 128), jnp.float32)],
    )(p0, p1, bias, Wsi, Asi, Adi)


def _tc_final_call(p0, p1, bias, batch3, Wfc, bfc):
    grid = NP // RB
    res = pl.pallas_call(
        _tc_final,
        grid=(grid,),
        in_specs=[_row_spec(ROWW), _row_spec(ROWW), _full_spec((1, 128)),
                  pl.BlockSpec((1, 1, RB), lambda i: (i, 0, 0)),
                  _full_spec((128, 128)), _full_spec((1, 128))],
        out_specs=[_full_spec((NGROUPS, 128)), _full_spec((NGROUPS, 128)),
                   _full_spec((NGROUPS, 128))],
        out_shape=[jax.ShapeDtypeStruct((NGROUPS, 128), jnp.float32),
                   jax.ShapeDtypeStruct((NGROUPS, 128), jnp.float32),
                   jax.ShapeDtypeStruct((NGROUPS, 128), jnp.float32)],
    )(p0, p1, bias, batch3, Wfc, bfc)
    return res[2]


def _leaky(v):
    return jnp.where(v >= 0.0, v, 0.2 * v)


def _make_sc_edge(etot_pad):
    chunks_per_worker = etot_pad // (32 * K)
    edges_per_worker = chunks_per_worker * K
    rows_per_tile = NP // 16
    group = 54 if chunks_per_worker % 54 == 0 else (18 if chunks_per_worker % 18 == 0 else 2)
    n_groups = chunks_per_worker // group

    mesh = plsc.VectorSubcoreMesh(core_axis_name="c", subcore_axis_name="s")

    @functools.partial(
        pl.kernel,
        out_type=jax.ShapeDtypeStruct((2, NP, ROWW), jnp.float32),
        mesh=mesh,
        compiler_params=pltpu.CompilerParams(use_tc_tiling_on_sc=False),
        scratch_types=[
            pltpu.VMEM((group, K), jnp.int32),
            pltpu.VMEM((group, K), jnp.int32),
            pltpu.VMEM((K, 16), jnp.float32), pltpu.VMEM((K, 16), jnp.float32),
            pltpu.VMEM((K, 16), jnp.float32), pltpu.VMEM((K, 16), jnp.float32),
            pltpu.VMEM((K, 128), jnp.float32), pltpu.VMEM((K, 128), jnp.float32),
            pltpu.VMEM((K, ROWW), jnp.float32),
            pltpu.VMEM((16,), jnp.float32),
            pltpu.VMEM_SHARED((NP, ROWW), jnp.float32),
            pltpu.SemaphoreType.DMA, pltpu.SemaphoreType.DMA,
        ],
    )
    def sc_edge(src_hbm, dst_hbm, ts_hbm, td_hbm, xl_hbm, g_hbm, out_hbm,
                sbuf, dbuf, srows0, srows1, drows0, drows1,
                xrows0, xrows1, stag, gv, acc, sem0, sem1):
        cid = lax.axis_index("c")
        sid = lax.axis_index("s")
        wid = sid * 2 + cid
        base_row = sid * rows_per_tile
        base_chunk = wid * chunks_per_worker
        srows = (srows0, srows1)
        drows = (drows0, drows1)
        xrows = (xrows0, xrows1)
        sem = (sem0, sem1)

        @plsc.parallel_loop(0, K, unroll=4)
        def _(r):
            for cc in range(ROWW // 16):
                stag[r, cc * 16:(cc + 1) * 16] = jnp.zeros((16,), jnp.float32)

        for j in range(rows_per_tile // K):
            pltpu.sync_copy(stag, acc.at[pl.ds(base_row + j * K, K)])
        pltpu.sync_copy(g_hbm, gv)
        plsc.subcore_barrier()

        def launch(j, b):
            pltpu.async_copy(ts_hbm.at[sbuf.at[j]], srows[b], sem[b])
            pltpu.async_copy(td_hbm.at[dbuf.at[j]], drows[b], sem[b])
            pltpu.async_copy(xl_hbm.at[sbuf.at[j]], xrows[b], sem[b])

        def drain(j, b):
            pltpu.make_async_copy(ts_hbm.at[sbuf.at[j]], srows[b], sem[b]).wait()
            pltpu.make_async_copy(td_hbm.at[dbuf.at[j]], drows[b], sem[b]).wait()
            pltpu.make_async_copy(xl_hbm.at[sbuf.at[j]], xrows[b], sem[b]).wait()

        def compute(j, b):
            gvec = gv[...]

            @plsc.parallel_loop(0, K, unroll=8)
            def _(e):
                sv = srows[b][e, 0:16]
                dv = drows[b][e, 0:16]
                w = jnp.exp(_leaky(sv + dv) - _leaky(gvec + dv))
                stag[e, 128:144] = w
                for h in range(HEADS):
                    stag[e, h * 16:(h + 1) * 16] = (
                        xrows[b][e, h * 16:(h + 1) * 16] * w[h])

            pltpu.sync_copy(stag, acc.at[dbuf.at[j]], add=True)

        def group_body(gr, _):
            gbase = base_chunk + gr * group
            pltpu.sync_copy(src_hbm.at[pl.ds(gbase, group)], sbuf)
            pltpu.sync_copy(dst_hbm.at[pl.ds(gbase, group)], dbuf)
            launch(0, 0)

            def pair(jp, _):
                j0 = jp * 2
                launch(j0 + 1, 1)
                drain(j0, 0)
                compute(j0, 0)

                @pl.when(j0 + 2 < group)
                def _():
                    launch(j0 + 2, 0)

                drain(j0 + 1, 1)
                compute(j0 + 1, 1)
                return 0

            lax.fori_loop(0, group // 2, pair, 0)
            return 0

        lax.fori_loop(0, n_groups, group_body, 0)

        plsc.subcore_barrier()
        pltpu.sync_copy(acc.at[pl.ds(base_row, rows_per_tile)],
                        out_hbm.at[cid].at[pl.ds(base_row, rows_per_tile)])

    return sc_edge


def kernel(x, edge_index, batch, W0, b0, Ws, att_src, att_dst, conv_bias,
           Wfc, bfc):
    L = Ws.shape[0]
    n = x.shape[0]

    xp = jnp.pad(x, ((0, NP - n), (0, 0)))
    loop = jnp.arange(n, dtype=edge_index.dtype)
    src = jnp.concatenate([edge_index[0], loop])
    dst = jnp.concatenate([edge_index[1], loop])
    etot = src.shape[0]
    etot_pad = ((etot + EDGE_ALIGN - 1) // EDGE_ALIGN) * EDGE_ALIGN
    src = jnp.pad(src, (0, etot_pad - etot)).reshape(-1, K)
    dst = jnp.pad(dst, (0, etot_pad - etot),
                  constant_values=DUMP).reshape(-1, K)
    batch3 = jnp.pad(batch, (0, NP - n), constant_values=-1).reshape(
        NP // RB, 1, RB)

    eye = jnp.eye(HEADS, dtype=jnp.float32)
    As = (att_src[:, :, :, None] * eye[:, None, :]).reshape(L, HID, HEADS)
    Ad = (att_dst[:, :, :, None] * eye[:, None, :]).reshape(L, HID, HEADS)
    As = jnp.pad(As, ((0, 0), (0, 0), (0, 8)))
    Ad = jnp.pad(Ad, ((0, 0), (0, 0), (0, 8)))
    b0r = b0.reshape(1, HID)
    biasr = conv_bias.reshape(L, 1, HID)
    bfcr = bfc.reshape(1, -1)

    sc_edge = _make_sc_edge(etot_pad)

    def g_fix(g_out):
        g16 = g_out[0, :16]
        return jnp.where(jnp.arange(16) < 8, g16, 1e9).astype(jnp.float32)

    xl, ts, td, g = _tc_first_call(xp, W0, b0r, Ws[0], As[0], Ad[0])
    parts = sc_edge(src, dst, ts, td, xl, g_fix(g))
    for i in range(1, L):
        xl, ts, td, g = _tc_mid_call(parts[0], parts[1], biasr[i - 1],
                                     Ws[i], As[i], Ad[i])
        parts = sc_edge(src, dst, ts, td, xl, g_fix(g))
    return _tc_final_call(parts[0], parts[1], biasr[L - 1], batch3, Wfc, bfcr)
